# Initial kernel scaffold; baseline (speedup 1.0000x reference)
import functools

import jax
import jax.numpy as jnp
from jax import lax
from jax.experimental import pallas as pl
from jax.experimental.pallas import tpu as pltpu
from jax.experimental.pallas import tpu_sc as plsc

_NC = 2
_NS = 16
_NW = _NC * _NS
_L = 16
_C = 128


def _sc_sampled_mm(rows_p, cols_p, mask_p, mat1, mat2t, steps):
    padded = rows_p.shape[0]
    per_tile = steps * _C
    kd = mat1.shape[1]
    mesh = plsc.VectorSubcoreMesh(core_axis_name="c", subcore_axis_name="s")

    @functools.partial(
        pl.kernel,
        out_type=jax.ShapeDtypeStruct((padded,), jnp.float32),
        mesh=mesh,
        scratch_types=[
            pltpu.VMEM((_C,), jnp.int32),
            pltpu.VMEM((_C,), jnp.int32),
            pltpu.VMEM((_C,), jnp.float32),
            pltpu.VMEM((_C, 64), jnp.float32),
            pltpu.VMEM((_C, 64), jnp.float32),
            pltpu.VMEM((_L, _L), jnp.float32),
            pltpu.VMEM((_C,), jnp.float32),
            pltpu.SemaphoreType.DMA,
            pltpu.SemaphoreType.DMA,
        ],
    )
    def k(rows_hbm, cols_hbm, mask_hbm, mat1_hbm, mat2t_hbm, out_hbm,
          ridx, cidx, mvals, g1, g2, accm, obuf, sem1, sem2):
        wid = lax.axis_index("s") * _NC + lax.axis_index("c")
        base = wid * per_tile

        @pl.loop(0, per_tile, step=_C)
        def _chunk(off):
            start = base + off
            pltpu.sync_copy(rows_hbm.at[pl.ds(start, _C)], ridx)
            pltpu.sync_copy(cols_hbm.at[pl.ds(start, _C)], cidx)
            pltpu.sync_copy(mask_hbm.at[pl.ds(start, _C)], mvals)
            c1 = pltpu.async_copy(mat1_hbm.at[ridx], g1, sem1)
            c2 = pltpu.async_copy(mat2t_hbm.at[cidx], g2, sem2)
            c1.wait()
            c2.wait()

            @pl.loop(0, _C, step=_L)
            def _block(b):
                for j in range(_L):
                    p = b + j
                    acc = g1[p, pl.ds(0, _L)] * g2[p, pl.ds(0, _L)]
                    for kk in range(1, 64 // _L):
                        acc = acc + (g1[p, pl.ds(kk * _L, _L)] *
                                     g2[p, pl.ds(kk * _L, _L)])
                    accm[j, :] = acc
                iot = lax.iota(jnp.int32, _L)
                tot = plsc.load_gather(accm, [iot, jnp.zeros((_L,), jnp.int32)])
                for cix in range(1, _L):
                    col = jnp.full((_L,), cix, jnp.int32)
                    tot = tot + plsc.load_gather(accm, [iot, col])
                obuf[pl.ds(b, _L)] = tot + mvals[pl.ds(b, _L)]

            pltpu.sync_copy(obuf, out_hbm.at[pl.ds(start, _C)])

    return k(rows_p, cols_p, mask_p, mat1, mat2t)


def kernel(rows, cols, mask_vals, mat1, mat2):
    nnz = rows.shape[0]
    grain = _NW * _C
    steps = -(-nnz // grain)
    pad = steps * grain - nnz
    rows_p = jnp.pad(rows, (0, pad))
    cols_p = jnp.pad(cols, (0, pad))
    mask_p = jnp.pad(mask_vals, (0, pad))
    out = _sc_sampled_mm(rows_p, cols_p, mask_p, mat1, mat2.T, steps)
    return out[:nnz]


# speedup vs baseline: 2.9534x; 2.9534x over previous
import dataclasses
import functools

import jax
import jax.numpy as jnp
from jax import lax
from jax.experimental import pallas as pl
from jax.experimental.pallas import tpu as pltpu
from jax.experimental.pallas import tpu_sc as plsc

_NC = 2
_NS = 16
_NW = _NC * _NS
_L = 16
_C = 128


def _sc_sampled_mm(rows_p, cols_p, mask_p, mat1, mat2t, steps):
    padded = rows_p.shape[0]
    per_tile = steps * _C
    kd = mat1.shape[1]
    mesh = plsc.VectorSubcoreMesh(core_axis_name="c", subcore_axis_name="s")
    cp = pltpu.CompilerParams()
    if "needs_layout_passes" in pltpu.CompilerParams.__dataclass_fields__:
        cp = dataclasses.replace(cp, needs_layout_passes=False)
    if "use_tc_tiling_on_sc" in pltpu.CompilerParams.__dataclass_fields__:
        cp = dataclasses.replace(cp, use_tc_tiling_on_sc=False)

    @functools.partial(
        pl.kernel,
        compiler_params=cp,
        out_type=jax.ShapeDtypeStruct((padded,), jnp.float32),
        mesh=mesh,
        scratch_types=[
            pltpu.VMEM((_C,), jnp.int32),
            pltpu.VMEM((_C,), jnp.int32),
            pltpu.VMEM((_C,), jnp.float32),
            pltpu.VMEM((_C, 64), jnp.float32),
            pltpu.VMEM((_C, 64), jnp.float32),
            pltpu.VMEM((_L, _L), jnp.float32),
            pltpu.VMEM((_C,), jnp.float32),
            pltpu.SemaphoreType.DMA,
            pltpu.SemaphoreType.DMA,
        ],
    )
    def k(rows_hbm, cols_hbm, mask_hbm, mat1_hbm, mat2t_hbm, out_hbm,
          ridx, cidx, mvals, g1, g2, accm, obuf, sem1, sem2):
        wid = lax.axis_index("s") * _NC + lax.axis_index("c")
        base = wid * per_tile

        @pl.loop(0, per_tile, step=_C)
        def _chunk(off):
            start = base + off
            pltpu.sync_copy(rows_hbm.at[pl.ds(start, _C)], ridx)
            pltpu.sync_copy(cols_hbm.at[pl.ds(start, _C)], cidx)
            pltpu.sync_copy(mask_hbm.at[pl.ds(start, _C)], mvals)
            c1 = pltpu.async_copy(mat1_hbm.at[ridx], g1, sem1)
            c2 = pltpu.async_copy(mat2t_hbm.at[cidx], g2, sem2)
            c1.wait()
            c2.wait()

            @pl.loop(0, _C, step=_L)
            def _block(b):
                for j in range(_L):
                    p = b + j
                    acc = g1[p, pl.ds(0, _L)] * g2[p, pl.ds(0, _L)]
                    for kk in range(1, 64 // _L):
                        acc = acc + (g1[p, pl.ds(kk * _L, _L)] *
                                     g2[p, pl.ds(kk * _L, _L)])
                    accm[j, :] = acc
                iot = lax.iota(jnp.int32, _L)
                tot = plsc.load_gather(accm, [iot, jnp.zeros((_L,), jnp.int32)])
                for cix in range(1, _L):
                    col = jnp.full((_L,), cix, jnp.int32)
                    tot = tot + plsc.load_gather(accm, [iot, col])
                obuf[pl.ds(b, _L)] = tot + mvals[pl.ds(b, _L)]

            pltpu.sync_copy(obuf, out_hbm.at[pl.ds(start, _C)])

    return k(rows_p, cols_p, mask_p, mat1, mat2t)


def kernel(rows, cols, mask_vals, mat1, mat2):
    nnz = rows.shape[0]
    grain = _NW * _C
    steps = -(-nnz // grain)
    pad = steps * grain - nnz
    rows_p = jnp.pad(rows, (0, pad))
    cols_p = jnp.pad(cols, (0, pad))
    mask_p = jnp.pad(mask_vals, (0, pad))
    out = _sc_sampled_mm(rows_p, cols_p, mask_p, mat1, mat2.T, steps)
    return out[:nnz]


# speedup vs baseline: 3.2990x; 1.1170x over previous
import dataclasses
import functools

import jax
import jax.numpy as jnp
from jax import lax
from jax.experimental import pallas as pl
from jax.experimental.pallas import tpu as pltpu
from jax.experimental.pallas import tpu_sc as plsc

_NC = 2
_NS = 16
_NW = _NC * _NS
_L = 16
_C = 256
_G = 128
_K = 64


def _sc_sampled_mm(rows_p, cols_p, mask_p, mat1, mat2t, steps):
    padded = rows_p.shape[0]
    per_tile = steps * _C
    mesh = plsc.VectorSubcoreMesh(core_axis_name="c", subcore_axis_name="s")
    cp = pltpu.CompilerParams()
    if "needs_layout_passes" in pltpu.CompilerParams.__dataclass_fields__:
        cp = dataclasses.replace(cp, needs_layout_passes=False)
    if "use_tc_tiling_on_sc" in pltpu.CompilerParams.__dataclass_fields__:
        cp = dataclasses.replace(cp, use_tc_tiling_on_sc=False)

    @functools.partial(
        pl.kernel,
        compiler_params=cp,
        out_type=jax.ShapeDtypeStruct((padded,), jnp.float32),
        mesh=mesh,
        scratch_types=[
            pltpu.VMEM((per_tile,), jnp.int32),
            pltpu.VMEM((per_tile,), jnp.int32),
            pltpu.VMEM((per_tile,), jnp.float32),
            pltpu.VMEM((per_tile,), jnp.float32),
            pltpu.VMEM((_C, _K), jnp.float32),
            pltpu.VMEM((_C, _K), jnp.float32),
            pltpu.VMEM((_C, _K), jnp.float32),
            pltpu.VMEM((_C, _K), jnp.float32),
            pltpu.VMEM((_L, _L), jnp.float32),
            pltpu.SemaphoreType.DMA,
            pltpu.SemaphoreType.DMA,
        ],
    )
    def k(rows_hbm, cols_hbm, mask_hbm, mat1_hbm, mat2t_hbm, out_hbm,
          ridx, cidx, mval, obuf, g1a, g1b, g2a, g2b, accm, sem0, sem1):
        wid = lax.axis_index("s") * _NC + lax.axis_index("c")
        base = wid * per_tile

        pltpu.sync_copy(rows_hbm.at[pl.ds(base, per_tile)], ridx)
        pltpu.sync_copy(cols_hbm.at[pl.ds(base, per_tile)], cidx)
        pltpu.sync_copy(mask_hbm.at[pl.ds(base, per_tile)], mval)

        g1 = (g1a, g1b)
        g2 = (g2a, g2b)
        sems = (sem0, sem1)

        def fire(c, b):
            off = c * _C
            for i in range(_C // _G):
                pltpu.async_copy(
                    mat1_hbm.at[ridx.at[pl.ds(off + i * _G, _G)]],
                    g1[b].at[pl.ds(i * _G, _G), :], sems[b])
                pltpu.async_copy(
                    mat2t_hbm.at[cidx.at[pl.ds(off + i * _G, _G)]],
                    g2[b].at[pl.ds(i * _G, _G), :], sems[b])

        def drain(b):
            for i in range(_C // _G):
                pltpu.make_async_copy(
                    mat1_hbm.at[ridx.at[pl.ds(i * _G, _G)]],
                    g1[b].at[pl.ds(i * _G, _G), :], sems[b]).wait()
                pltpu.make_async_copy(
                    mat2t_hbm.at[cidx.at[pl.ds(i * _G, _G)]],
                    g2[b].at[pl.ds(i * _G, _G), :], sems[b]).wait()

        def compute(c, b):
            off = c * _C
            g1r, g2r = g1[b], g2[b]

            @pl.loop(0, _C, step=_L)
            def _block(bb):
                for j in range(_L):
                    p = bb + j
                    acc = g1r[p, pl.ds(0, _L)] * g2r[p, pl.ds(0, _L)]
                    for kk in range(1, _K // _L):
                        acc = acc + (g1r[p, pl.ds(kk * _L, _L)] *
                                     g2r[p, pl.ds(kk * _L, _L)])
                    accm[j, :] = acc
                iot = lax.iota(jnp.int32, _L)
                tot = plsc.load_gather(accm, [iot, jnp.zeros((_L,), jnp.int32)])
                for cix in range(1, _L):
                    col = jnp.full((_L,), cix, jnp.int32)
                    tot = tot + plsc.load_gather(accm, [iot, col])
                obuf[pl.ds(off + bb, _L)] = tot + mval[pl.ds(off + bb, _L)]

        fire(0, 0)

        @pl.loop(0, steps - 2, step=2)
        def _pipe(s0):
            fire(s0 + 1, 1)
            drain(0)
            compute(s0, 0)
            fire(s0 + 2, 0)
            drain(1)
            compute(s0 + 1, 1)

        fire(steps - 1, 1)
        drain(0)
        compute(steps - 2, 0)
        drain(1)
        compute(steps - 1, 1)

        pltpu.sync_copy(obuf, out_hbm.at[pl.ds(base, per_tile)])

    return k(rows_p, cols_p, mask_p, mat1, mat2t)


def kernel(rows, cols, mask_vals, mat1, mat2):
    nnz = rows.shape[0]
    grain = _NW * _C
    steps = -(-nnz // grain)
    if steps % 2:
        steps += 1
    pad = steps * grain - nnz
    rows_p = jnp.pad(rows, (0, pad))
    cols_p = jnp.pad(cols, (0, pad))
    mask_p = jnp.pad(mask_vals, (0, pad))
    out = _sc_sampled_mm(rows_p, cols_p, mask_p, mat1, mat2.T, steps)
    return out[:nnz]


# speedup vs baseline: 5.6296x; 1.7064x over previous
import dataclasses
import functools

import jax
import jax.numpy as jnp
from jax import lax
from jax.experimental import pallas as pl
from jax.experimental.pallas import tpu as pltpu
from jax.experimental.pallas import tpu_sc as plsc

_NC = 2
_NS = 16
_NW = _NC * _NS
_L = 16
_C = 256
_G = 128
_K = 64


def _sc_sampled_mm(rows_p, cols_p, mask_p, mat1, mat2t, steps):
    padded = rows_p.shape[0]
    per_tile = steps * _C
    mesh = plsc.VectorSubcoreMesh(core_axis_name="c", subcore_axis_name="s")
    cp = pltpu.CompilerParams()
    if "needs_layout_passes" in pltpu.CompilerParams.__dataclass_fields__:
        cp = dataclasses.replace(cp, needs_layout_passes=False)
    if "use_tc_tiling_on_sc" in pltpu.CompilerParams.__dataclass_fields__:
        cp = dataclasses.replace(cp, use_tc_tiling_on_sc=False)

    @functools.partial(
        pl.kernel,
        compiler_params=cp,
        out_type=jax.ShapeDtypeStruct((padded,), jnp.float32),
        mesh=mesh,
        scratch_types=[
            pltpu.VMEM((per_tile,), jnp.int32),
            pltpu.VMEM((per_tile,), jnp.int32),
            pltpu.VMEM((per_tile,), jnp.float32),
            pltpu.VMEM((per_tile,), jnp.float32),
            pltpu.VMEM((_C, _K), jnp.bfloat16),
            pltpu.VMEM((_C, _K), jnp.bfloat16),
            pltpu.VMEM((_C, _K), jnp.bfloat16),
            pltpu.VMEM((_C, _K), jnp.bfloat16),
            pltpu.VMEM((_L, _L), jnp.float32),
            pltpu.SemaphoreType.DMA,
            pltpu.SemaphoreType.DMA,
        ],
    )
    def k(rows_hbm, cols_hbm, mask_hbm, mat1_hbm, mat2t_hbm, out_hbm,
          ridx, cidx, mval, obuf, g1a, g1b, g2a, g2b, accm, sem0, sem1):
        wid = lax.axis_index("s") * _NC + lax.axis_index("c")
        base = wid * per_tile

        pltpu.sync_copy(rows_hbm.at[pl.ds(base, per_tile)], ridx)
        pltpu.sync_copy(cols_hbm.at[pl.ds(base, per_tile)], cidx)
        pltpu.sync_copy(mask_hbm.at[pl.ds(base, per_tile)], mval)

        g1 = (g1a, g1b)
        g2 = (g2a, g2b)
        sems = (sem0, sem1)

        def fire(c, b):
            off = c * _C
            for i in range(_C // _G):
                pltpu.async_copy(
                    mat1_hbm.at[ridx.at[pl.ds(off + i * _G, _G)]],
                    g1[b].at[pl.ds(i * _G, _G), :], sems[b])
                pltpu.async_copy(
                    mat2t_hbm.at[cidx.at[pl.ds(off + i * _G, _G)]],
                    g2[b].at[pl.ds(i * _G, _G), :], sems[b])

        def drain(b):
            for i in range(_C // _G):
                pltpu.make_async_copy(
                    mat1_hbm.at[ridx.at[pl.ds(i * _G, _G)]],
                    g1[b].at[pl.ds(i * _G, _G), :], sems[b]).wait()
                pltpu.make_async_copy(
                    mat2t_hbm.at[cidx.at[pl.ds(i * _G, _G)]],
                    g2[b].at[pl.ds(i * _G, _G), :], sems[b]).wait()

        def compute(c, b):
            off = c * _C
            g1r, g2r = g1[b], g2[b]

            @pl.loop(0, _C, step=_L)
            def _block(bb):
                for j in range(_L):
                    p = bb + j
                    acc = None
                    for kk in range(_K // (2 * _L)):
                        prod = (g1r[p, pl.ds(kk * 2 * _L, 2 * _L)] *
                                g2r[p, pl.ds(kk * 2 * _L, 2 * _L)])
                        lo, hi = plsc.unpack(
                            prod, format=plsc.PackFormat.INTERLEAVED,
                            preferred_element_type=jnp.float32)
                        half = lo + hi
                        acc = half if acc is None else acc + half
                    accm[j, :] = acc
                iot = lax.iota(jnp.int32, _L)
                tot = plsc.load_gather(accm, [iot, jnp.zeros((_L,), jnp.int32)])
                for cix in range(1, _L):
                    col = jnp.full((_L,), cix, jnp.int32)
                    tot = tot + plsc.load_gather(accm, [iot, col])
                obuf[pl.ds(off + bb, _L)] = tot + mval[pl.ds(off + bb, _L)]

        fire(0, 0)

        @pl.loop(0, steps - 2, step=2)
        def _pipe(s0):
            fire(s0 + 1, 1)
            drain(0)
            compute(s0, 0)
            fire(s0 + 2, 0)
            drain(1)
            compute(s0 + 1, 1)

        fire(steps - 1, 1)
        drain(0)
        compute(steps - 2, 0)
        drain(1)
        compute(steps - 1, 1)

        pltpu.sync_copy(obuf, out_hbm.at[pl.ds(base, per_tile)])

    return k(rows_p, cols_p, mask_p, mat1, mat2t)


def kernel(rows, cols, mask_vals, mat1, mat2):
    nnz = rows.shape[0]
    grain = _NW * _C
    steps = -(-nnz // grain)
    if steps % 2:
        steps += 1
    pad = steps * grain - nnz
    rows_p = jnp.pad(rows, (0, pad))
    cols_p = jnp.pad(cols, (0, pad))
    mask_p = jnp.pad(mask_vals, (0, pad))
    out = _sc_sampled_mm(rows_p, cols_p, mask_p,
                         mat1.astype(jnp.bfloat16),
                         mat2.T.astype(jnp.bfloat16), steps)
    return out[:nnz]


# speedup vs baseline: 5.7880x; 1.0281x over previous
import dataclasses
import functools

import jax
import jax.numpy as jnp
from jax import lax
from jax.experimental import pallas as pl
from jax.experimental.pallas import tpu as pltpu
from jax.experimental.pallas import tpu_sc as plsc

_NC = 2
_NS = 16
_NW = _NC * _NS
_L = 16
_C = 128
_NB = 4
_K = 64


def _sc_sampled_mm(rows_p, cols_p, mask_p, mat1, mat2t, steps):
    padded = rows_p.shape[0]
    per_tile = steps * _C
    mesh = plsc.VectorSubcoreMesh(core_axis_name="c", subcore_axis_name="s")
    cp = pltpu.CompilerParams()
    if "needs_layout_passes" in pltpu.CompilerParams.__dataclass_fields__:
        cp = dataclasses.replace(cp, needs_layout_passes=False)
    if "use_tc_tiling_on_sc" in pltpu.CompilerParams.__dataclass_fields__:
        cp = dataclasses.replace(cp, use_tc_tiling_on_sc=False)

    @functools.partial(
        pl.kernel,
        compiler_params=cp,
        out_type=jax.ShapeDtypeStruct((padded,), jnp.float32),
        mesh=mesh,
        scratch_types=[
            pltpu.VMEM((per_tile,), jnp.int32),
            pltpu.VMEM((per_tile,), jnp.int32),
            pltpu.VMEM((per_tile,), jnp.float32),
            pltpu.VMEM((per_tile,), jnp.float32),
            *[pltpu.VMEM((_C, _K), jnp.bfloat16)
              for _ in range(_NB)],
            *[pltpu.VMEM((_C, _K), jnp.bfloat16)
              for _ in range(_NB)],
            pltpu.VMEM((_L, _L), jnp.float32),
            *[pltpu.SemaphoreType.DMA for _ in range(_NB)],
        ],
    )
    def k(rows_hbm, cols_hbm, mask_hbm, mat1_hbm, mat2t_hbm, out_hbm,
          ridx, cidx, mval, obuf, *rest):
        g1 = rest[:_NB]
        g2 = rest[_NB:2 * _NB]
        accm = rest[2 * _NB]
        sems = rest[2 * _NB + 1:]
        wid = lax.axis_index("s") * _NC + lax.axis_index("c")
        base = wid * per_tile

        pltpu.sync_copy(rows_hbm.at[pl.ds(base, per_tile)], ridx)
        pltpu.sync_copy(cols_hbm.at[pl.ds(base, per_tile)], cidx)
        pltpu.sync_copy(mask_hbm.at[pl.ds(base, per_tile)], mval)

        def fire(c, b):
            off = c * _C
            pltpu.async_copy(
                mat1_hbm.at[ridx.at[pl.ds(off, _C)]], g1[b], sems[b])
            pltpu.async_copy(
                mat2t_hbm.at[cidx.at[pl.ds(off, _C)]], g2[b], sems[b])

        def drain(b):
            pltpu.make_async_copy(
                mat1_hbm.at[ridx.at[pl.ds(0, _C)]], g1[b], sems[b]).wait()
            pltpu.make_async_copy(
                mat2t_hbm.at[cidx.at[pl.ds(0, _C)]], g2[b], sems[b]).wait()

        def compute(c, b):
            off = c * _C
            g1r, g2r = g1[b], g2[b]

            @pl.loop(0, _C, step=_L)
            def _block(bb):
                for j in range(_L):
                    p = bb + j
                    acc = None
                    for kk in range(_K // (2 * _L)):
                        prod = (g1r[p, pl.ds(kk * 2 * _L, 2 * _L)] *
                                g2r[p, pl.ds(kk * 2 * _L, 2 * _L)])
                        lo, hi = plsc.unpack(
                            prod, format=plsc.PackFormat.INTERLEAVED,
                            preferred_element_type=jnp.float32)
                        half = lo + hi
                        acc = half if acc is None else acc + half
                    accm[j, :] = acc
                iot = lax.iota(jnp.int32, _L)
                tot = plsc.load_gather(accm, [iot, jnp.zeros((_L,), jnp.int32)])
                for cix in range(1, _L):
                    col = jnp.full((_L,), cix, jnp.int32)
                    tot = tot + plsc.load_gather(accm, [iot, col])
                obuf[pl.ds(off + bb, _L)] = tot + mval[pl.ds(off + bb, _L)]

        for b in range(_NB):
            fire(b, b)

        @pl.loop(0, steps - _NB, step=_NB)
        def _pipe(s0):
            for b in range(_NB):
                drain(b)
                compute(s0 + b, b)
                fire(s0 + b + _NB, b)

        for b in range(_NB):
            drain(b)
            compute(steps - _NB + b, b)

        pltpu.sync_copy(obuf, out_hbm.at[pl.ds(base, per_tile)])

    return k(rows_p, cols_p, mask_p, mat1, mat2t)


def kernel(rows, cols, mask_vals, mat1, mat2):
    nnz = rows.shape[0]
    grain = _NW * _C
    steps = -(-nnz // grain)
    steps = -(-steps // _NB) * _NB
    pad = steps * grain - nnz
    rows_p = jnp.pad(rows, (0, pad))
    cols_p = jnp.pad(cols, (0, pad))
    mask_p = jnp.pad(mask_vals, (0, pad))
    out = _sc_sampled_mm(rows_p, cols_p, mask_p,
                         mat1.astype(jnp.bfloat16),
                         mat2.T.astype(jnp.bfloat16), steps)
    return out[:nnz]
